# Initial kernel scaffold; baseline (speedup 1.0000x reference)
import jax
import jax.numpy as jnp
from jax.experimental import pallas as pl


def kernel(x, Wg, bg, W1, b1, W2, b2):
    raise NotImplementedError("write your pallas kernel here")


# speedup vs baseline: 3.0362x; 3.0362x over previous
import functools

import jax
import jax.numpy as jnp
from jax.experimental import pallas as pl
from jax.experimental.pallas import tpu as pltpu

E = 8
TOP = 2
D = 768
N = 2048
GATE_BLK = 256


def _gate_body(x_ref, wg_ref, bg_ref, wmat_ref):
    x = x_ref[...]
    logits = jnp.dot(x, wg_ref[...], preferred_element_type=jnp.float32)
    logits = logits + bg_ref[...]
    m = jnp.max(logits, axis=1, keepdims=True)
    p = jnp.exp(logits - m)
    p = p / jnp.sum(p, axis=1, keepdims=True)
    ii = jax.lax.broadcasted_iota(jnp.int32, p.shape, 1)
    m1 = jnp.max(p, axis=1, keepdims=True)
    i1 = jnp.min(jnp.where(p == m1, ii, E), axis=1, keepdims=True)
    p2 = jnp.where(ii == i1, -1.0, p)
    m2 = jnp.max(p2, axis=1, keepdims=True)
    i2 = jnp.min(jnp.where(p2 == m2, ii, E), axis=1, keepdims=True)
    r = jnp.exp(m2 - m1)
    w1 = 1.0 / (1.0 + r)
    w2 = 1.0 - w1
    wmat_ref[...] = (
        w1 * (ii == i1).astype(jnp.float32) + w2 * (ii == i2).astype(jnp.float32)
    )


def _moe_body(x_ref, w1_ref, b1_ref, w2_ref, b2_ref, wmat_ref, out_ref):
    e = pl.program_id(0)
    x = x_ref[...]
    h = jnp.dot(x, w1_ref[0], preferred_element_type=jnp.float32) + b1_ref[0]
    h = jnp.maximum(h, 0.0)
    ya = jnp.dot(h, w2_ref[0], preferred_element_type=jnp.float32) + b2_ref[0]
    onehot = (jax.lax.broadcasted_iota(jnp.int32, (E, 1), 0) == e).astype(
        jnp.float32
    )
    wcol = jnp.dot(wmat_ref[...], onehot, preferred_element_type=jnp.float32)
    contrib = ya * wcol

    @pl.when(e == 0)
    def _():
        out_ref[...] = contrib

    @pl.when(e != 0)
    def _():
        out_ref[...] = out_ref[...] + contrib


def _gate(xr, Wg, bg):
    return pl.pallas_call(
        _gate_body,
        grid=(N // GATE_BLK,),
        in_specs=[
            pl.BlockSpec((GATE_BLK, D), lambda i: (i, 0)),
            pl.BlockSpec((D, E), lambda i: (0, 0)),
            pl.BlockSpec((1, E), lambda i: (0, 0)),
        ],
        out_specs=pl.BlockSpec((GATE_BLK, E), lambda i: (i, 0)),
        out_shape=jax.ShapeDtypeStruct((N, E), jnp.float32),
    )(xr, Wg, bg.reshape(1, E))


def _moe(xr, W1, b1, W2, b2, wmat):
    return pl.pallas_call(
        _moe_body,
        grid=(E,),
        in_specs=[
            pl.BlockSpec((N, D), lambda e: (0, 0)),
            pl.BlockSpec((1, D, D), lambda e: (e, 0, 0)),
            pl.BlockSpec((1, 1, D), lambda e: (e, 0, 0)),
            pl.BlockSpec((1, D, D), lambda e: (e, 0, 0)),
            pl.BlockSpec((1, 1, D), lambda e: (e, 0, 0)),
            pl.BlockSpec((N, E), lambda e: (0, 0)),
        ],
        out_specs=pl.BlockSpec((N, D), lambda e: (0, 0)),
        out_shape=jax.ShapeDtypeStruct((N, D), jnp.float32),
        compiler_params=pltpu.CompilerParams(
            dimension_semantics=("arbitrary",),
        ),
    )(xr, W1, b1.reshape(E, 1, D), W2, b2.reshape(E, 1, D), wmat)


@jax.jit
def kernel(x, Wg, bg, W1, b1, W2, b2):
    x_shape = x.shape
    xr = x.reshape(-1, D)
    wmat = _gate(xr, Wg, bg)
    y = _moe(xr, W1, b1, W2, b2, wmat)
    return y.reshape(x_shape)
